# baseline (speedup 1.0000x reference)
import functools

import jax
import jax.numpy as jnp
from jax import lax
from jax.experimental import pallas as pl
from jax.experimental.pallas import tpu as pltpu
from jax.experimental.pallas import tpu_sc as plsc


@functools.lru_cache(maxsize=None)
def _build_gather(R, D):
    nc = plsc.get_sparse_core_info().num_cores
    mesh = plsc.VectorSubcoreMesh(core_axis_name="c", subcore_axis_name="s")

    @functools.partial(
        pl.kernel,
        mesh=mesh,
        out_type=jax.ShapeDtypeStruct((R, D), jnp.float32),
        scratch_types=[
            pltpu.VMEM((R,), jnp.int32),
            pltpu.VMEM((R, D), jnp.float32),
            pltpu.SemaphoreType.DMA,
        ],
    )
    def sc_gather(idx_hbm, emb_hbm, out_hbm, idx_v, rows_v, sem):
        wid = lax.axis_index("s") * nc + lax.axis_index("c")

        @pl.when(wid == 0)
        def _():
            pltpu.sync_copy(idx_hbm, idx_v)
            pltpu.async_copy(emb_hbm.at[idx_v], rows_v, sem).wait()
            pltpu.sync_copy(rows_v, out_hbm)

    return sc_gather


@functools.lru_cache(maxsize=None)
def _build_concat(B, C, D, R):
    def body(x_ref, e_ref, o_ref):
        r = pl.program_id(0)

        @pl.when(r < R)
        def _():
            o_ref[0] = jnp.broadcast_to(e_ref[...].reshape(1, D), (B, D))

        @pl.when(r >= R)
        def _():
            o_ref[0] = jnp.swapaxes(x_ref[0], 0, 1)

    return pl.pallas_call(
        body,
        grid=(R + C,),
        in_specs=[
            pl.BlockSpec((1, D, B), lambda r: (jnp.maximum(r - R, 0), 0, 0)),
            pl.BlockSpec((1, 1, D), lambda r: (jnp.minimum(r, R - 1), 0, 0)),
        ],
        out_specs=pl.BlockSpec((1, B, D), lambda r: (r, 0, 0)),
        out_shape=jax.ShapeDtypeStruct((R + C, B, D), jnp.float32),
        compiler_params=pltpu.CompilerParams(
            dimension_semantics=("arbitrary",),
        ),
    )


def kernel(x, y, emb_table):
    B, C = x.shape[0], x.shape[1]
    R, D = emb_table.shape
    xT = jnp.transpose(x.reshape(B, C, D), (1, 2, 0))
    idx = y.reshape(-1).astype(jnp.int32)
    emb_rows = _build_gather(R, D)(idx, emb_table)
    outT = _build_concat(B, C, D, R)(xT, emb_rows.reshape(R, 1, D))
    return jnp.transpose(outT, (1, 0, 2))


# speedup vs baseline: 1.5449x; 1.5449x over previous
import functools

import jax
import jax.numpy as jnp
from jax import lax
from jax.experimental import pallas as pl
from jax.experimental.pallas import tpu as pltpu
from jax.experimental.pallas import tpu_sc as plsc


@functools.lru_cache(maxsize=None)
def _build_gather(R, D):
    nc = plsc.get_sparse_core_info().num_cores
    mesh = plsc.VectorSubcoreMesh(core_axis_name="c", subcore_axis_name="s")

    @functools.partial(
        pl.kernel,
        mesh=mesh,
        out_type=jax.ShapeDtypeStruct((R, D), jnp.float32),
        scratch_types=[
            pltpu.VMEM((R,), jnp.int32),
            pltpu.VMEM((R, D), jnp.float32),
            pltpu.SemaphoreType.DMA,
        ],
    )
    def sc_gather(idx_hbm, emb_hbm, out_hbm, idx_v, rows_v, sem):
        wid = lax.axis_index("s") * nc + lax.axis_index("c")

        @pl.when(wid == 0)
        def _():
            pltpu.sync_copy(idx_hbm, idx_v)
            pltpu.async_copy(emb_hbm.at[idx_v], rows_v, sem).wait()
            pltpu.sync_copy(rows_v, out_hbm)

    return sc_gather


@functools.lru_cache(maxsize=None)
def _build_concat(B, C, D, R, rows_per_step=4):
    nsteps = -(-(R + C) // rows_per_step)

    def body(*refs):
        x_refs = refs[:rows_per_step]
        e_refs = refs[rows_per_step : 2 * rows_per_step]
        o_ref = refs[2 * rows_per_step]
        r = pl.program_id(0)
        ident = (
            lax.broadcasted_iota(jnp.int32, (B, B), 0)
            == lax.broadcasted_iota(jnp.int32, (B, B), 1)
        ).astype(jnp.float32)
        for k in range(rows_per_step):
            j = r * rows_per_step + k

            @pl.when(j < R)
            def _(k=k):
                o_ref[k] = lax.dot_general(
                    jnp.full((B, 1), 1.0, dtype=jnp.float32),
                    e_refs[k][0],
                    (((1,), (0,)), ((), ())),
                    precision=lax.Precision.HIGHEST,
                    preferred_element_type=jnp.float32,
                )

            @pl.when(j >= R)
            def _(k=k):
                o_ref[k] = lax.dot_general(
                    ident,
                    x_refs[k][0],
                    (((1,), (1,)), ((), ())),
                    precision=lax.Precision.HIGHEST,
                    preferred_element_type=jnp.float32,
                )

    def x_map(k):
        return lambda r: (
            jnp.clip(r * rows_per_step + k - R, 0, C - 1),
            0,
            0,
        )

    def e_map(k):
        return lambda r: (jnp.minimum(r * rows_per_step + k, R - 1), 0, 0)

    return pl.pallas_call(
        body,
        grid=(nsteps,),
        in_specs=[pl.BlockSpec((1, D, B), x_map(k)) for k in range(rows_per_step)]
        + [pl.BlockSpec((1, 1, D), e_map(k)) for k in range(rows_per_step)],
        out_specs=pl.BlockSpec((rows_per_step, B, D), lambda r: (r, 0, 0)),
        out_shape=jax.ShapeDtypeStruct((R + C, B, D), jnp.float32),
        compiler_params=pltpu.CompilerParams(
            dimension_semantics=("arbitrary",),
        ),
    )


def kernel(x, y, emb_table):
    B, C = x.shape[0], x.shape[1]
    R, D = emb_table.shape
    xT = jnp.transpose(x.reshape(B, C, D), (1, 2, 0))
    idx = y.reshape(-1).astype(jnp.int32)
    emb_rows = _build_gather(R, D)(idx, emb_table)
    rows_per_step = 4
    e3 = emb_rows.reshape(R, 1, D)
    outT = _build_concat(B, C, D, R, rows_per_step)(
        *([xT] * rows_per_step), *([e3] * rows_per_step)
    )
    return jnp.transpose(outT, (1, 0, 2))


# speedup vs baseline: 1.5497x; 1.0031x over previous
import functools

import jax
import jax.numpy as jnp
from jax import lax
from jax.experimental import pallas as pl
from jax.experimental.pallas import tpu as pltpu
from jax.experimental.pallas import tpu_sc as plsc


@functools.lru_cache(maxsize=None)
def _build_gather(R, D):
    nc = plsc.get_sparse_core_info().num_cores
    mesh = plsc.VectorSubcoreMesh(core_axis_name="c", subcore_axis_name="s")

    @functools.partial(
        pl.kernel,
        mesh=mesh,
        out_type=jax.ShapeDtypeStruct((R, D), jnp.float32),
        scratch_types=[
            pltpu.VMEM((R,), jnp.int32),
            pltpu.VMEM((R, D), jnp.float32),
            pltpu.SemaphoreType.DMA,
        ],
    )
    def sc_gather(idx_hbm, emb_hbm, out_hbm, idx_v, rows_v, sem):
        wid = lax.axis_index("s") * nc + lax.axis_index("c")

        @pl.when(wid == 0)
        def _():
            pltpu.sync_copy(idx_hbm, idx_v)
            pltpu.async_copy(emb_hbm.at[idx_v], rows_v, sem).wait()
            pltpu.sync_copy(rows_v, out_hbm)

    return sc_gather


@functools.lru_cache(maxsize=None)
def _build_concat(B, C, D, R, rows_per_step=8):
    nsteps = -(-(R + C) // rows_per_step)
    ne = min(rows_per_step, R)

    def body(*refs):
        x_refs = refs[:rows_per_step]
        e_refs = refs[rows_per_step : rows_per_step + ne]
        o_ref = refs[rows_per_step + ne]
        r = pl.program_id(0)
        ident = (
            lax.broadcasted_iota(jnp.int32, (B, B), 0)
            == lax.broadcasted_iota(jnp.int32, (B, B), 1)
        ).astype(jnp.float32)
        for k in range(rows_per_step):
            j = r * rows_per_step + k

            if k < ne:

                @pl.when(j < R)
                def _(k=k):
                    o_ref[k] = lax.dot_general(
                        jnp.full((B, 1), 1.0, dtype=jnp.float32),
                        e_refs[k][0],
                        (((1,), (0,)), ((), ())),
                        precision=lax.Precision.HIGHEST,
                        preferred_element_type=jnp.float32,
                    )

            @pl.when(j >= R)
            def _(k=k):
                o_ref[k] = lax.dot_general(
                    ident,
                    x_refs[k][0],
                    (((1,), (1,)), ((), ())),
                    precision=lax.Precision.HIGHEST,
                    preferred_element_type=jnp.float32,
                )

    def x_map(k):
        return lambda r: (
            jnp.clip(r * rows_per_step + k - R, 0, C - 1),
            0,
            0,
        )

    def e_map(k):
        return lambda r: (jnp.minimum(r * rows_per_step + k, R - 1), 0, 0)

    return pl.pallas_call(
        body,
        grid=(nsteps,),
        in_specs=[pl.BlockSpec((1, D, B), x_map(k)) for k in range(rows_per_step)]
        + [pl.BlockSpec((1, 1, D), e_map(k)) for k in range(ne)],
        out_specs=pl.BlockSpec((rows_per_step, B, D), lambda r: (r, 0, 0)),
        out_shape=jax.ShapeDtypeStruct((R + C, B, D), jnp.float32),
        compiler_params=pltpu.CompilerParams(
            dimension_semantics=("arbitrary",),
        ),
    )


def kernel(x, y, emb_table):
    B, C = x.shape[0], x.shape[1]
    R, D = emb_table.shape
    xT = jnp.transpose(x.reshape(B, C, D), (1, 2, 0))
    idx = y.reshape(-1).astype(jnp.int32)
    emb_rows = _build_gather(R, D)(idx, emb_table)
    rows_per_step = 8
    e3 = emb_rows.reshape(R, 1, D)
    outT = _build_concat(B, C, D, R, rows_per_step)(
        *([xT] * rows_per_step), *([e3] * min(rows_per_step, R))
    )
    return jnp.transpose(outT, (1, 0, 2))
